# baseline (speedup 1.0000x reference)
import functools

import jax
import jax.numpy as jnp
from jax import lax
from jax.experimental import pallas as pl
from jax.experimental.pallas import tpu as pltpu
from jax.experimental.pallas import tpu_sc as plsc

N_NODES = 50000
N_EDGES = 800000
NF = 64
RBF = 32

_NC = 2
_NS = 16
_NW = _NC * _NS

_BLK = 6400
_NBLK = N_EDGES // _BLK

_NSEG = 5
_BPS = _NBLK // _NSEG
_EPS = N_EDGES // _NSEG

_SUB = 320
_PPB = _BLK // 2
_SPB = _PPB // _SUB
_JOBS_SEG = (_EPS // 2) // _SUB


def _node_proj_body(ns_t_ref, w_ref, p_ref):
    p_ref[...] = lax.dot_general(
        ns_t_ref[...], w_ref[...],
        dimension_numbers=(((0,), (0,)), ((), ())),
        preferred_element_type=jnp.float32,
        precision=lax.Precision.DEFAULT,
    )


def _node_proj(ns_t, w_sd):
    return pl.pallas_call(
        _node_proj_body,
        out_shape=jax.ShapeDtypeStruct((N_NODES, 2 * NF), jnp.float32),
    )(ns_t, w_sd)


def _gather_add_body(seg, tab_hbm, src_hbm, dst_hbm, out_hbm,
                     silo, dilo, sihi, dihi, bla, blb, bha, bhb, bufo,
                     s1, s2, s3, s4):
    wid = lax.axis_index("s") * _NC + lax.axis_index("c")
    njobs = (_JOBS_SEG - wid + _NW - 1) // _NW

    def job(k, carry):
        j = wid + k * _NW
        b = j // _SPB
        s = j % _SPB
        lo = (seg * _BPS + b) * _BLK + s * _SUB
        hi = lo + _PPB
        r0 = b * _PPB + s * _SUB
        pltpu.sync_copy(src_hbm.at[pl.ds(lo, _SUB)], silo)
        pltpu.sync_copy(dst_hbm.at[pl.ds(lo, _SUB)], dilo)
        pltpu.sync_copy(src_hbm.at[pl.ds(hi, _SUB)], sihi)
        pltpu.sync_copy(dst_hbm.at[pl.ds(hi, _SUB)], dihi)
        c1 = pltpu.async_copy(tab_hbm.at[silo], bla, s1)
        c2 = pltpu.async_copy(tab_hbm.at[dilo], blb, s2)
        c3 = pltpu.async_copy(tab_hbm.at[sihi], bha, s3)
        c4 = pltpu.async_copy(tab_hbm.at[dihi], bhb, s4)
        c1.wait()
        c2.wait()
        c3.wait()
        c4.wait()

        def add_row(i, c2_):
            for jj in range(4):
                sl = pl.ds(jj * 16, 16)
                bufo[i, sl] = bla[i, sl] + blb[i, sl]
                bufo[i, pl.ds(NF + jj * 16, 16)] = bha[i, sl] + bhb[i, sl]
            return c2_

        lax.fori_loop(0, _SUB, add_row, 0)
        pltpu.sync_copy(bufo, out_hbm.at[pl.ds(r0, _SUB)])
        return carry

    lax.fori_loop(0, njobs, job, 0)


def _gather_add_seg(seg, table, src2, dst2):
    mesh = plsc.VectorSubcoreMesh(core_axis_name="c", subcore_axis_name="s")
    fn = pl.kernel(
        functools.partial(_gather_add_body, seg),
        mesh=mesh,
        compiler_params=pltpu.CompilerParams(use_tc_tiling_on_sc=False),
        out_type=jax.ShapeDtypeStruct((_EPS // 2, 2 * NF), jnp.float32),
        scratch_types=[
            pltpu.VMEM((_SUB,), jnp.int32),
            pltpu.VMEM((_SUB,), jnp.int32),
            pltpu.VMEM((_SUB,), jnp.int32),
            pltpu.VMEM((_SUB,), jnp.int32),
            pltpu.VMEM((_SUB, NF), jnp.float32),
            pltpu.VMEM((_SUB, NF), jnp.float32),
            pltpu.VMEM((_SUB, NF), jnp.float32),
            pltpu.VMEM((_SUB, NF), jnp.float32),
            pltpu.VMEM((_SUB, 2 * NF), jnp.float32),
            pltpu.SemaphoreType.DMA,
            pltpu.SemaphoreType.DMA,
            pltpu.SemaphoreType.DMA,
            pltpu.SemaphoreType.DMA,
        ],
    )
    return fn(table, src2, dst2)


def _edge_mlp_t_body_aliased(prev_ref, *refs):
    del prev_ref
    _edge_mlp_t_body(*refs)


def _edge_mlp_t_body(g_ref, ef_ref, d_ref, w1e_t_ref, w1r_t_ref,
                     b1_ref, w2_t_ref, b2_ref, gam_ref, bet_ref, out_ref):
    ef = ef_ref[...]
    gt = jnp.transpose(g_ref[...])
    g_t = jnp.concatenate([gt[:NF, :], gt[NF:, :]], axis=1)
    h = (
        g_t
        + jnp.dot(w1e_t_ref[...], ef, preferred_element_type=jnp.float32,
                  precision=lax.Precision.DEFAULT)
        + jnp.dot(w1r_t_ref[...], d_ref[...], preferred_element_type=jnp.float32,
                  precision=lax.Precision.DEFAULT)
        + b1_ref[...]
    )
    h = h * jax.nn.sigmoid(h)
    h = jnp.dot(w2_t_ref[...], h, preferred_element_type=jnp.float32,
                precision=lax.Precision.DEFAULT) + b2_ref[...]
    h = h * jax.nn.sigmoid(h)
    y = ef + h
    mean = jnp.mean(y, axis=0, keepdims=True)
    var = jnp.mean(jnp.square(y - mean), axis=0, keepdims=True)
    out_ref[...] = (y - mean) * lax.rsqrt(var + 1e-5) * gam_ref[...] + bet_ref[...]


def _edge_mlp_t_seg(seg, prev, g128, ef_t, d_t, w1e_t, w1r_t, b1c, w2_t,
                    b2c, gam_c, bet_c):
    cst = lambda i: (0, 0)
    off = seg * _BPS
    specs = [
        pl.BlockSpec((_BLK // 2, 2 * NF), lambda i: (i, 0)),
        pl.BlockSpec((NF, _BLK), lambda i, off=off: (0, off + i)),
        pl.BlockSpec((RBF, _BLK), lambda i, off=off: (0, off + i)),
        pl.BlockSpec((NF, NF), cst),
        pl.BlockSpec((NF, RBF), cst),
        pl.BlockSpec((NF, 1), cst),
        pl.BlockSpec((NF, NF), cst),
        pl.BlockSpec((NF, 1), cst),
        pl.BlockSpec((NF, 1), cst),
        pl.BlockSpec((NF, 1), cst),
    ]
    args = (g128, ef_t, d_t, w1e_t, w1r_t, b1c, w2_t, b2c, gam_c, bet_c)
    body = _edge_mlp_t_body
    aliases = {}
    if prev is not None:
        specs = [pl.BlockSpec(memory_space=pl.ANY)] + specs
        args = (prev,) + args
        body = _edge_mlp_t_body_aliased
        aliases = {0: 0}
    return pl.pallas_call(
        body,
        grid=(_BPS,),
        in_specs=specs,
        out_specs=pl.BlockSpec((NF, _BLK), lambda i, off=off: (0, off + i)),
        out_shape=jax.ShapeDtypeStruct((NF, N_EDGES), jnp.float32),
        input_output_aliases=aliases,
    )(*args)


def kernel(node_scalars, edge_feats, d, src_idxs, dst_idxs,
           W1, b1, W2, b2, ln_gamma, ln_beta):
    w_sd = jnp.concatenate([W1[:NF], W1[NF:2 * NF]], axis=1)
    w1e_t = W1[2 * NF:3 * NF].T
    w1r_t = W1[3 * NF:].T

    src2 = src_idxs.astype(jnp.int32) * 2
    dst2 = dst_idxs.astype(jnp.int32) * 2 + 1

    p = _node_proj(node_scalars.T, w_sd)
    table = p.reshape(2 * N_NODES, NF)

    ef_t, d_t = edge_feats.T, d.T
    b1c, b2c = b1.reshape(NF, 1), b2.reshape(NF, 1)
    gam_c, bet_c = ln_gamma.reshape(NF, 1), ln_beta.reshape(NF, 1)

    gs = [_gather_add_seg(seg, table, src2, dst2) for seg in range(_NSEG)]
    out_t = None
    for seg in range(_NSEG):
        out_t = _edge_mlp_t_seg(seg, out_t, gs[seg], ef_t, d_t, w1e_t, w1r_t,
                                b1c, W2.T, b2c, gam_c, bet_c)
    return out_t.T


# speedup vs baseline: 1.4886x; 1.4886x over previous
import functools

import jax
import jax.numpy as jnp
from jax import lax
from jax.experimental import pallas as pl
from jax.experimental.pallas import tpu as pltpu
from jax.experimental.pallas import tpu_sc as plsc

N_NODES = 50000
N_EDGES = 800000
NF = 64
RBF = 32

_NC = 2
_NS = 16
_NW = _NC * _NS

_BLK = 6400
_NBLK = N_EDGES // _BLK

_SUB = 320
_PPB = _BLK // 2
_SPB = _PPB // _SUB


def _node_proj_body(ns_t_ref, w_ref, p_ref):
    p_ref[...] = lax.dot_general(
        ns_t_ref[...], w_ref[...],
        dimension_numbers=(((0,), (0,)), ((), ())),
        preferred_element_type=jnp.float32,
        precision=lax.Precision.DEFAULT,
    )


def _node_proj(ns_t, w_sd):
    return pl.pallas_call(
        _node_proj_body,
        out_shape=jax.ShapeDtypeStruct((N_NODES, 2 * NF), jnp.float32),
    )(ns_t, w_sd)


def _gather_add_body(tab_hbm, src_hbm, dst_hbm, out_hbm,
                     sslab, dslab, bla, blb, bha, bhb,
                     s1, s2, s3, s4):
    wid = lax.axis_index("s") * _NC + lax.axis_index("c")
    nblk_w = (_NBLK - wid + _NW - 1) // _NW

    def block(kb, carry):
        b = wid + kb * _NW
        e0 = b * _BLK
        pltpu.sync_copy(src_hbm.at[pl.ds(e0, _BLK)], sslab)
        pltpu.sync_copy(dst_hbm.at[pl.ds(e0, _BLK)], dslab)

        def job(s, c_):
            r0 = b * _PPB + s * _SUB
            c1 = pltpu.async_copy(
                tab_hbm.at[sslab.at[pl.ds(s * _SUB, _SUB)]], bla, s1)
            c2 = pltpu.async_copy(
                tab_hbm.at[dslab.at[pl.ds(s * _SUB, _SUB)]], blb, s2)
            c3 = pltpu.async_copy(
                tab_hbm.at[sslab.at[pl.ds(_PPB + s * _SUB, _SUB)]], bha, s3)
            c4 = pltpu.async_copy(
                tab_hbm.at[dslab.at[pl.ds(_PPB + s * _SUB, _SUB)]], bhb, s4)
            c1.wait()
            c2.wait()
            c3.wait()
            c4.wait()

            def add_row(i, c2_):
                for jj in range(4):
                    sl = pl.ds(jj * 16, 16)
                    bla[i, sl] = bla[i, sl] + blb[i, sl]
                    bha[i, sl] = bha[i, sl] + bhb[i, sl]
                return c2_

            lax.fori_loop(0, _SUB, add_row, 0)
            pltpu.sync_copy(bla, out_hbm.at[pl.ds(r0, _SUB), pl.ds(0, NF)])
            pltpu.sync_copy(bha, out_hbm.at[pl.ds(r0, _SUB), pl.ds(NF, NF)])
            return c_

        lax.fori_loop(0, _SPB, job, 0)
        return carry

    lax.fori_loop(0, nblk_w, block, 0)


def _gather_add(table, src2, dst2):
    mesh = plsc.VectorSubcoreMesh(core_axis_name="c", subcore_axis_name="s")
    fn = pl.kernel(
        _gather_add_body,
        mesh=mesh,
        compiler_params=pltpu.CompilerParams(use_tc_tiling_on_sc=False),
        out_type=jax.ShapeDtypeStruct((N_EDGES // 2, 2 * NF), jnp.float32),
        scratch_types=[
            pltpu.VMEM((_BLK,), jnp.int32),
            pltpu.VMEM((_BLK,), jnp.int32),
            pltpu.VMEM((_SUB, NF), jnp.float32),
            pltpu.VMEM((_SUB, NF), jnp.float32),
            pltpu.VMEM((_SUB, NF), jnp.float32),
            pltpu.VMEM((_SUB, NF), jnp.float32),
            pltpu.SemaphoreType.DMA,
            pltpu.SemaphoreType.DMA,
            pltpu.SemaphoreType.DMA,
            pltpu.SemaphoreType.DMA,
        ],
    )
    return fn(table, src2, dst2)


def _edge_mlp_t_body(g_ref, ef_ref, d_ref, w1e_t_ref, w1r_t_ref,
                     b1_ref, w2_t_ref, b2_ref, gam_ref, bet_ref, out_ref):
    ef = ef_ref[...]
    gt = jnp.transpose(g_ref[...])
    g_t = jnp.concatenate([gt[:NF, :], gt[NF:, :]], axis=1)
    h = (
        g_t
        + jnp.dot(w1e_t_ref[...], ef, preferred_element_type=jnp.float32,
                  precision=lax.Precision.DEFAULT)
        + jnp.dot(w1r_t_ref[...], d_ref[...], preferred_element_type=jnp.float32,
                  precision=lax.Precision.DEFAULT)
        + b1_ref[...]
    )
    h = h * jax.nn.sigmoid(h)
    h = jnp.dot(w2_t_ref[...], h, preferred_element_type=jnp.float32,
                precision=lax.Precision.DEFAULT) + b2_ref[...]
    h = h * jax.nn.sigmoid(h)
    y = ef + h
    mean = jnp.mean(y, axis=0, keepdims=True)
    var = jnp.mean(jnp.square(y - mean), axis=0, keepdims=True)
    out_ref[...] = (y - mean) * lax.rsqrt(var + 1e-5) * gam_ref[...] + bet_ref[...]


def _edge_mlp_t(g128, ef_t, d_t, w1e_t, w1r_t, b1c, w2_t, b2c, gam_c, bet_c):
    cst = lambda i: (0, 0)
    return pl.pallas_call(
        _edge_mlp_t_body,
        grid=(_NBLK,),
        in_specs=[
            pl.BlockSpec((_BLK // 2, 2 * NF), lambda i: (i, 0)),
            pl.BlockSpec((NF, _BLK), lambda i: (0, i)),
            pl.BlockSpec((RBF, _BLK), lambda i: (0, i)),
            pl.BlockSpec((NF, NF), cst),
            pl.BlockSpec((NF, RBF), cst),
            pl.BlockSpec((NF, 1), cst),
            pl.BlockSpec((NF, NF), cst),
            pl.BlockSpec((NF, 1), cst),
            pl.BlockSpec((NF, 1), cst),
            pl.BlockSpec((NF, 1), cst),
        ],
        out_specs=pl.BlockSpec((NF, _BLK), lambda i: (0, i)),
        out_shape=jax.ShapeDtypeStruct((NF, N_EDGES), jnp.float32),
    )(g128, ef_t, d_t, w1e_t, w1r_t, b1c, w2_t, b2c, gam_c, bet_c)


def kernel(node_scalars, edge_feats, d, src_idxs, dst_idxs,
           W1, b1, W2, b2, ln_gamma, ln_beta):
    w_sd = jnp.concatenate([W1[:NF], W1[NF:2 * NF]], axis=1)
    w1e_t = W1[2 * NF:3 * NF].T
    w1r_t = W1[3 * NF:].T

    src2 = src_idxs.astype(jnp.int32) * 2
    dst2 = dst_idxs.astype(jnp.int32) * 2 + 1

    p = _node_proj(node_scalars.T, w_sd)
    table = p.reshape(2 * N_NODES, NF)

    ef_t, d_t = edge_feats.T, d.T
    b1c, b2c = b1.reshape(NF, 1), b2.reshape(NF, 1)
    gam_c, bet_c = ln_gamma.reshape(NF, 1), ln_beta.reshape(NF, 1)

    g128 = _gather_add(table, src2, dst2)
    out_t = _edge_mlp_t(g128, ef_t, d_t, w1e_t, w1r_t,
                        b1c, W2.T, b2c, gam_c, bet_c)
    return out_t.T


# speedup vs baseline: 1.8156x; 1.2197x over previous
import functools

import jax
import jax.numpy as jnp
import numpy as np
from jax import lax
from jax.experimental import pallas as pl
from jax.experimental.pallas import tpu as pltpu
from jax.experimental.pallas import tpu_sc as plsc

N_NODES = 50000
N_EDGES = 800000
NF = 64
RBF = 32

_NC = 2
_NS = 16
_NW = _NC * _NS

_BLK = 6400
_NBLK = N_EDGES // _BLK

_SUB = 200
_PPB = _BLK // 2
_SPB = _PPB // _SUB


def _node_proj_body(ns_t_ref, w_ref, p_ref):
    p_ref[...] = lax.dot_general(
        ns_t_ref[...], w_ref[...],
        dimension_numbers=(((0,), (0,)), ((), ())),
        preferred_element_type=jnp.float32,
        precision=lax.Precision.DEFAULT,
    )


def _node_proj(ns_t, w_sd):
    return pl.pallas_call(
        _node_proj_body,
        out_shape=jax.ShapeDtypeStruct((N_NODES, 2 * NF), jnp.float32),
    )(ns_t, w_sd)


def _gather_add_body(tab_hbm, src_hbm, dst_hbm, out_hbm,
                     sslab, dslab,
                     bla0, blb0, bha0, bhb0, bla1, blb1, bha1, bhb1,
                     s10, s20, s30, s40, s11, s21, s31, s41):
    wid = lax.axis_index("s") * _NC + lax.axis_index("c")
    nblk_w = (_NBLK - wid + _NW - 1) // _NW
    sets = (
        (bla0, blb0, bha0, bhb0, s10, s20, s30, s40),
        (bla1, blb1, bha1, bhb1, s11, s21, s31, s41),
    )

    def issue(bufs, s):
        bla, blb, bha, bhb, s1, s2, s3, s4 = bufs
        c1 = pltpu.async_copy(
            tab_hbm.at[sslab.at[pl.ds(s * _SUB, _SUB)]], bla, s1)
        c2 = pltpu.async_copy(
            tab_hbm.at[dslab.at[pl.ds(s * _SUB, _SUB)]], blb, s2)
        c3 = pltpu.async_copy(
            tab_hbm.at[sslab.at[pl.ds(_PPB + s * _SUB, _SUB)]], bha, s3)
        c4 = pltpu.async_copy(
            tab_hbm.at[dslab.at[pl.ds(_PPB + s * _SUB, _SUB)]], bhb, s4)
        return (c1, c2, c3, c4)

    def process(bufs, b, s):
        bla, blb, bha, bhb, s1, s2, s3, s4 = bufs
        pltpu.make_async_copy(
            tab_hbm.at[sslab.at[pl.ds(0, _SUB)]], bla, s1).wait()
        pltpu.make_async_copy(
            tab_hbm.at[sslab.at[pl.ds(0, _SUB)]], blb, s2).wait()
        pltpu.make_async_copy(
            tab_hbm.at[sslab.at[pl.ds(0, _SUB)]], bha, s3).wait()
        pltpu.make_async_copy(
            tab_hbm.at[sslab.at[pl.ds(0, _SUB)]], bhb, s4).wait()

        def add_row(i, c2_):
            for jj in range(4):
                sl = pl.ds(jj * 16, 16)
                bla[i, sl] = bla[i, sl] + blb[i, sl]
                bha[i, sl] = bha[i, sl] + bhb[i, sl]
            return c2_

        lax.fori_loop(0, _SUB, add_row, 0)
        r0 = b * _PPB + s * _SUB
        pltpu.sync_copy(bla, out_hbm.at[pl.ds(r0, _SUB), pl.ds(0, NF)])
        pltpu.sync_copy(bha, out_hbm.at[pl.ds(r0, _SUB), pl.ds(NF, NF)])

    def block(kb, carry):
        b = wid + kb * _NW
        e0 = b * _BLK
        pltpu.sync_copy(src_hbm.at[pl.ds(e0, _BLK)], sslab)
        pltpu.sync_copy(dst_hbm.at[pl.ds(e0, _BLK)], dslab)
        issue(sets[0], 0)

        def pair(s2_, c_):
            s = 2 * s2_
            issue(sets[1], s + 1)
            process(sets[0], b, s)
            issue(sets[0], s + 2)
            process(sets[1], b, s + 1)
            return c_

        lax.fori_loop(0, _SPB // 2 - 1, pair, 0)
        issue(sets[1], _SPB - 1)
        process(sets[0], b, _SPB - 2)
        process(sets[1], b, _SPB - 1)
        return carry

    lax.fori_loop(0, nblk_w, block, 0)


def _gather_add(table, src2, dst2):
    mesh = plsc.VectorSubcoreMesh(core_axis_name="c", subcore_axis_name="s")
    fn = pl.kernel(
        _gather_add_body,
        mesh=mesh,
        compiler_params=pltpu.CompilerParams(use_tc_tiling_on_sc=False),
        out_type=jax.ShapeDtypeStruct((N_EDGES // 2, 2 * NF), jnp.float32),
        scratch_types=(
            [pltpu.VMEM((_BLK,), jnp.int32)] * 2
            + [pltpu.VMEM((_SUB, NF), jnp.float32)] * 8
            + [pltpu.SemaphoreType.DMA] * 8
        ),
    )
    return fn(table, src2, dst2)


def _edge_mlp_t_body(g_ref, ef_ref, d_ref, w1e_t_ref, w1r_t_ref,
                     b1_ref, w2_t_ref, b2_ref, gam_ref, bet_ref, out_ref):
    ef = ef_ref[...]
    gt = jnp.transpose(g_ref[...])
    g_t = jnp.concatenate([gt[:NF, :], gt[NF:, :]], axis=1)
    h = (
        g_t
        + jnp.dot(w1e_t_ref[...], ef, preferred_element_type=jnp.float32,
                  precision=lax.Precision.DEFAULT)
        + jnp.dot(w1r_t_ref[...], d_ref[...], preferred_element_type=jnp.float32,
                  precision=lax.Precision.DEFAULT)
        + b1_ref[...]
    )
    h = h * jax.nn.sigmoid(h)
    h = jnp.dot(w2_t_ref[...], h, preferred_element_type=jnp.float32,
                precision=lax.Precision.DEFAULT) + b2_ref[...]
    h = h * jax.nn.sigmoid(h)
    y = ef + h
    mean = jnp.mean(y, axis=0, keepdims=True)
    var = jnp.mean(jnp.square(y - mean), axis=0, keepdims=True)
    out_ref[...] = (y - mean) * lax.rsqrt(var + 1e-5) * gam_ref[...] + bet_ref[...]


def _edge_mlp_t(g128, ef_t, d_t, w1e_t, w1r_t, b1c, w2_t, b2c, gam_c, bet_c):
    cst = lambda i: (0, 0)
    return pl.pallas_call(
        _edge_mlp_t_body,
        grid=(_NBLK,),
        in_specs=[
            pl.BlockSpec((_BLK // 2, 2 * NF), lambda i: (i, 0)),
            pl.BlockSpec((NF, _BLK), lambda i: (0, i)),
            pl.BlockSpec((RBF, _BLK), lambda i: (0, i)),
            pl.BlockSpec((NF, NF), cst),
            pl.BlockSpec((NF, RBF), cst),
            pl.BlockSpec((NF, 1), cst),
            pl.BlockSpec((NF, NF), cst),
            pl.BlockSpec((NF, 1), cst),
            pl.BlockSpec((NF, 1), cst),
            pl.BlockSpec((NF, 1), cst),
        ],
        out_specs=pl.BlockSpec((NF, _BLK), lambda i: (0, i)),
        out_shape=jax.ShapeDtypeStruct((NF, N_EDGES), jnp.float32),
    )(g128, ef_t, d_t, w1e_t, w1r_t, b1c, w2_t, b2c, gam_c, bet_c)


def kernel(node_scalars, edge_feats, d, src_idxs, dst_idxs,
           W1, b1, W2, b2, ln_gamma, ln_beta):
    w_sd = jnp.concatenate([W1[:NF], W1[NF:2 * NF]], axis=1)
    w1e_t = W1[2 * NF:3 * NF].T
    w1r_t = W1[3 * NF:].T

    src2 = src_idxs.astype(jnp.int32) * 2
    dst2 = dst_idxs.astype(jnp.int32) * 2 + 1

    p = _node_proj(node_scalars.T, w_sd)
    table = p.reshape(2 * N_NODES, NF)

    ef_t, d_t = edge_feats.T, d.T
    b1c, b2c = b1.reshape(NF, 1), b2.reshape(NF, 1)
    gam_c, bet_c = ln_gamma.reshape(NF, 1), ln_beta.reshape(NF, 1)

    g128 = _gather_add(table, src2, dst2)
    out_t = _edge_mlp_t(g128, ef_t, d_t, w1e_t, w1r_t,
                        b1c, W2.T, b2c, gam_c, bet_c)
    return out_t.T


# speedup vs baseline: 1.9362x; 1.0664x over previous
import functools

import jax
import jax.numpy as jnp
import numpy as np
from jax import lax
from jax.experimental import pallas as pl
from jax.experimental.pallas import tpu as pltpu
from jax.experimental.pallas import tpu_sc as plsc

N_NODES = 50000
N_EDGES = 800000
NF = 64
RBF = 32

_NC = 2
_NS = 16
_NW = _NC * _NS

_BLK = 6400
_NBLK = N_EDGES // _BLK

_SUB = 200
_PPB = _BLK // 2
_SPB = _PPB // _SUB


def _node_proj_body(ns_t_ref, w_ref, p_ref):
    p_ref[...] = lax.dot_general(
        ns_t_ref[...], w_ref[...],
        dimension_numbers=(((0,), (0,)), ((), ())),
        preferred_element_type=jnp.float32,
        precision=lax.Precision.DEFAULT,
    )


def _node_proj(ns_t, w_sd):
    return pl.pallas_call(
        _node_proj_body,
        out_shape=jax.ShapeDtypeStruct((N_NODES, 2 * NF), jnp.float32),
    )(ns_t, w_sd)


def _gather_add_body(b0, nb, tab_hbm, src_hbm, dst_hbm, out_hbm,
                     sslab, dslab,
                     bla0, blb0, bha0, bhb0, bla1, blb1, bha1, bhb1,
                     s10, s20, s30, s40, s11, s21, s31, s41):
    wid = lax.axis_index("s") * _NC + lax.axis_index("c")
    nblk_w = (nb - wid + _NW - 1) // _NW
    sets = (
        (bla0, blb0, bha0, bhb0, s10, s20, s30, s40),
        (bla1, blb1, bha1, bhb1, s11, s21, s31, s41),
    )

    def issue(bufs, s):
        bla, blb, bha, bhb, s1, s2, s3, s4 = bufs
        c1 = pltpu.async_copy(
            tab_hbm.at[sslab.at[pl.ds(s * _SUB, _SUB)]], bla, s1)
        c2 = pltpu.async_copy(
            tab_hbm.at[dslab.at[pl.ds(s * _SUB, _SUB)]], blb, s2)
        c3 = pltpu.async_copy(
            tab_hbm.at[sslab.at[pl.ds(_PPB + s * _SUB, _SUB)]], bha, s3)
        c4 = pltpu.async_copy(
            tab_hbm.at[dslab.at[pl.ds(_PPB + s * _SUB, _SUB)]], bhb, s4)
        return (c1, c2, c3, c4)

    def process(bufs, b, s):
        bla, blb, bha, bhb, s1, s2, s3, s4 = bufs
        pltpu.make_async_copy(
            tab_hbm.at[sslab.at[pl.ds(0, _SUB)]], bla, s1).wait()
        pltpu.make_async_copy(
            tab_hbm.at[sslab.at[pl.ds(0, _SUB)]], blb, s2).wait()
        pltpu.make_async_copy(
            tab_hbm.at[sslab.at[pl.ds(0, _SUB)]], bha, s3).wait()
        pltpu.make_async_copy(
            tab_hbm.at[sslab.at[pl.ds(0, _SUB)]], bhb, s4).wait()

        def add_row(i, c2_):
            for jj in range(4):
                sl = pl.ds(jj * 16, 16)
                bla[i, sl] = bla[i, sl] + blb[i, sl]
                bha[i, sl] = bha[i, sl] + bhb[i, sl]
            return c2_

        lax.fori_loop(0, _SUB, add_row, 0)
        r0 = (b - b0) * _PPB + s * _SUB
        pltpu.sync_copy(bla, out_hbm.at[pl.ds(r0, _SUB), pl.ds(0, NF)])
        pltpu.sync_copy(bha, out_hbm.at[pl.ds(r0, _SUB), pl.ds(NF, NF)])

    def block(kb, carry):
        b = b0 + wid + kb * _NW
        e0 = b * _BLK
        pltpu.sync_copy(src_hbm.at[pl.ds(e0, _BLK)], sslab)
        pltpu.sync_copy(dst_hbm.at[pl.ds(e0, _BLK)], dslab)
        issue(sets[0], 0)

        def pair(s2_, c_):
            s = 2 * s2_
            issue(sets[1], s + 1)
            process(sets[0], b, s)
            issue(sets[0], s + 2)
            process(sets[1], b, s + 1)
            return c_

        lax.fori_loop(0, _SPB // 2 - 1, pair, 0)
        issue(sets[1], _SPB - 1)
        process(sets[0], b, _SPB - 2)
        process(sets[1], b, _SPB - 1)
        return carry

    lax.fori_loop(0, nblk_w, block, 0)


def _gather_add(table, src2, dst2, b0, nb):
    mesh = plsc.VectorSubcoreMesh(core_axis_name="c", subcore_axis_name="s")
    fn = pl.kernel(
        functools.partial(_gather_add_body, b0, nb),
        mesh=mesh,
        compiler_params=pltpu.CompilerParams(use_tc_tiling_on_sc=False),
        out_type=jax.ShapeDtypeStruct((nb * _PPB, 2 * NF), jnp.float32),
        scratch_types=(
            [pltpu.VMEM((_BLK,), jnp.int32)] * 2
            + [pltpu.VMEM((_SUB, NF), jnp.float32)] * 8
            + [pltpu.SemaphoreType.DMA] * 8
        ),
    )
    return fn(table, src2, dst2)


def _edge_mlp_t_body_aliased(prev_ref, *refs):
    del prev_ref
    _edge_mlp_t_body(*refs)


def _edge_mlp_t_body(g_ref, ef_ref, d_ref, w1e_t_ref, w1r_t_ref,
                     b1_ref, w2_t_ref, b2_ref, gam_ref, bet_ref, out_ref):
    ef = ef_ref[...]
    gt = jnp.transpose(g_ref[...])
    g_t = jnp.concatenate([gt[:NF, :], gt[NF:, :]], axis=1)
    h = (
        g_t
        + jnp.dot(w1e_t_ref[...], ef, preferred_element_type=jnp.float32,
                  precision=lax.Precision.DEFAULT)
        + jnp.dot(w1r_t_ref[...], d_ref[...], preferred_element_type=jnp.float32,
                  precision=lax.Precision.DEFAULT)
        + b1_ref[...]
    )
    h = h * jax.nn.sigmoid(h)
    h = jnp.dot(w2_t_ref[...], h, preferred_element_type=jnp.float32,
                precision=lax.Precision.DEFAULT) + b2_ref[...]
    h = h * jax.nn.sigmoid(h)
    y = ef + h
    mean = jnp.mean(y, axis=0, keepdims=True)
    var = jnp.mean(jnp.square(y - mean), axis=0, keepdims=True)
    out_ref[...] = (y - mean) * lax.rsqrt(var + 1e-5) * gam_ref[...] + bet_ref[...]


def _edge_mlp_t(prev, b0, nb, g128, ef_t, d_t, w1e_t, w1r_t, b1c, w2_t,
                b2c, gam_c, bet_c):
    cst = lambda i: (0, 0)
    specs = [
        pl.BlockSpec((_BLK // 2, 2 * NF), lambda i: (i, 0)),
        pl.BlockSpec((NF, _BLK), lambda i, b0=b0: (0, b0 + i)),
        pl.BlockSpec((RBF, _BLK), lambda i, b0=b0: (0, b0 + i)),
        pl.BlockSpec((NF, NF), cst),
        pl.BlockSpec((NF, RBF), cst),
        pl.BlockSpec((NF, 1), cst),
        pl.BlockSpec((NF, NF), cst),
        pl.BlockSpec((NF, 1), cst),
        pl.BlockSpec((NF, 1), cst),
        pl.BlockSpec((NF, 1), cst),
    ]
    args = (g128, ef_t, d_t, w1e_t, w1r_t, b1c, w2_t, b2c, gam_c, bet_c)
    body = _edge_mlp_t_body
    aliases = {}
    if prev is not None:
        specs = [pl.BlockSpec(memory_space=pl.ANY)] + specs
        args = (prev,) + args
        body = _edge_mlp_t_body_aliased
        aliases = {0: 0}
    return pl.pallas_call(
        body,
        grid=(nb,),
        in_specs=specs,
        out_specs=pl.BlockSpec((NF, _BLK), lambda i, b0=b0: (0, b0 + i)),
        out_shape=jax.ShapeDtypeStruct((NF, N_EDGES), jnp.float32),
        input_output_aliases=aliases,
    )(*args)


def kernel(node_scalars, edge_feats, d, src_idxs, dst_idxs,
           W1, b1, W2, b2, ln_gamma, ln_beta):
    w_sd = jnp.concatenate([W1[:NF], W1[NF:2 * NF]], axis=1)
    w1e_t = W1[2 * NF:3 * NF].T
    w1r_t = W1[3 * NF:].T

    src2 = src_idxs.astype(jnp.int32) * 2
    dst2 = dst_idxs.astype(jnp.int32) * 2 + 1

    p = _node_proj(node_scalars.T, w_sd)
    table = p.reshape(2 * N_NODES, NF)

    ef_t, d_t = edge_feats.T, d.T
    b1c, b2c = b1.reshape(NF, 1), b2.reshape(NF, 1)
    gam_c, bet_c = ln_gamma.reshape(NF, 1), ln_beta.reshape(NF, 1)

    segs = ((0, 63), (63, 62))
    gs = [_gather_add(table, src2, dst2, b0, nb) for b0, nb in segs]
    out_t = None
    for (b0, nb), g128 in zip(segs, gs):
        out_t = _edge_mlp_t(out_t, b0, nb, g128, ef_t, d_t, w1e_t, w1r_t,
                            b1c, W2.T, b2c, gam_c, bet_c)
    return out_t.T
